# Initial kernel scaffold; baseline (speedup 1.0000x reference)
import jax
import jax.numpy as jnp
from jax.experimental import pallas as pl


def kernel(scores):
    raise NotImplementedError("write your pallas kernel here")


# speedup vs baseline: 4.8641x; 4.8641x over previous
import functools

import jax
import jax.numpy as jnp
from jax import lax
from jax.experimental import pallas as pl
from jax.experimental.pallas import tpu as pltpu
from jax.experimental.pallas import tpu_sc as plsc

R = 64
N = 32768
K = 50
KPAD = 64
NC, NS, L = 2, 16, 16
NW = NC * NS
ROWS_PER_W = R // NW
NVEC = N // L
G1 = 16
NG1 = NVEC // G1
NG2 = NG1 // 16
CAP = 2048

_NEG_INF = float("-inf")


def _splat_f(x):
    return jnp.broadcast_to(jnp.float32(x), (L,))


def _splat_i(x):
    return jnp.broadcast_to(jnp.int32(x), (L,))


@functools.partial(
    pl.kernel,
    out_type=(
        jax.ShapeDtypeStruct((R, KPAD), jnp.int32),
        jax.ShapeDtypeStruct((R, KPAD), jnp.float32),
    ),
    mesh=plsc.VectorSubcoreMesh(core_axis_name="c", subcore_axis_name="s"),
    compiler_params=pltpu.CompilerParams(needs_layout_passes=False),
    scratch_types=[
        pltpu.VMEM((N,), jnp.float32),
        pltpu.VMEM((NG1 * L,), jnp.float32),
        pltpu.VMEM((CAP,), jnp.float32),
        pltpu.VMEM((CAP,), jnp.int32),
        pltpu.VMEM((KPAD,), jnp.float32),
        pltpu.VMEM((KPAD,), jnp.int32),
    ],
)
def _topk_sc(scores_hbm, oidx_hbm, ovals_hbm, row_v, gmax_v, cv_v, ci_v,
             sv_v, si_v):
    lane = lax.iota(jnp.int32, L)

    def do_row(r):
        pltpu.sync_copy(scores_hbm.at[r], row_v)

        def h_body(h, tcur):
            def g_body(i, g2acc):
                g = h * 16 + i

                def v_body(j, acc):
                    return jnp.maximum(acc, row_v[pl.ds((g * G1 + j) * L, L)])

                gm = lax.fori_loop(0, G1, v_body, _splat_f(-jnp.inf))
                gmax_v[pl.ds(g * L, L)] = gm
                return jnp.maximum(g2acc, gm)

            g2 = lax.fori_loop(0, 16, g_body, _splat_f(-jnp.inf))

            def r_body(_, x):
                return jnp.where(x == jnp.max(x), jnp.float32(_NEG_INF), x)

            t_h = jnp.max(lax.fori_loop(0, 6, r_body, g2))
            return jnp.minimum(tcur, t_h)

        t = lax.fori_loop(0, NG2, h_body, jnp.float32(jnp.inf))

        def z_body(i, _):
            cv_v[pl.ds(i * L, L)] = _splat_f(-jnp.inf)
            ci_v[pl.ds(i * L, L)] = _splat_i(2**30)
            return 0

        lax.fori_loop(0, CAP // L, z_body, 0)

        def c_body(i, off):
            v = row_v[pl.ds(i * L, L)]
            m = v >= t
            cnt = jnp.sum(m.astype(jnp.int32))
            pos = off + plsc.cumsum(m.astype(jnp.int32)) - 1
            pos = jnp.minimum(pos, CAP - 1)
            plsc.store_scatter(cv_v, [pos], v, mask=m)
            plsc.store_scatter(ci_v, [pos], i * L + lane, mask=m)
            return off + cnt

        m_total = lax.fori_loop(0, NVEC, c_body, 0)
        nvc = (jnp.minimum(m_total, CAP) + (L - 1)) // L

        def e_body(j, _):
            def mx_body(i, acc):
                return jnp.maximum(acc, cv_v[pl.ds(i * L, L)])

            mx = lax.fori_loop(0, nvc, mx_body, _splat_f(-jnp.inf))
            vstar = jnp.max(mx)

            def ix_body(i, acc):
                acc_idx, acc_pos = acc
                vv = cv_v[pl.ds(i * L, L)]
                ii = ci_v[pl.ds(i * L, L)]
                pp = i * L + lane
                better = (vv == vstar) & (ii < acc_idx)
                return (jnp.where(better, ii, acc_idx),
                        jnp.where(better, pp, acc_pos))

            acc_idx, acc_pos = lax.fori_loop(
                0, nvc, ix_body, (_splat_i(2**30), _splat_i(2**30)))
            istar = jnp.min(acc_idx)
            pstar = jnp.min(jnp.where(acc_idx == istar, acc_pos,
                                      jnp.int32(2**30)))
            lane0 = lane == 0
            plsc.store_scatter(sv_v, [_splat_i(0) + j],
                               jnp.broadcast_to(vstar, (L,)), mask=lane0)
            plsc.store_scatter(si_v, [_splat_i(0) + j],
                               jnp.broadcast_to(istar, (L,)), mask=lane0)
            plsc.store_scatter(cv_v, [_splat_i(0) + pstar],
                               _splat_f(-jnp.inf), mask=lane0)
            return 0

        lax.fori_loop(0, K, e_body, 0)

        pltpu.sync_copy(sv_v, ovals_hbm.at[r])
        pltpu.sync_copy(si_v, oidx_hbm.at[r])

    wid = lax.axis_index("s") * NC + lax.axis_index("c")
    for rr in range(ROWS_PER_W):
        do_row(wid * ROWS_PER_W + rr)


def kernel(scores):
    idx_pad, vals_pad = _topk_sc(scores)
    return idx_pad[:, :K], vals_pad[:, :K]


# speedup vs baseline: 5.7375x; 1.1796x over previous
import functools

import jax
import jax.numpy as jnp
from jax import lax
from jax.experimental import pallas as pl
from jax.experimental.pallas import tpu as pltpu
from jax.experimental.pallas import tpu_sc as plsc

R = 64
N = 32768
K = 50
KPAD = 64
NC, NS, L = 2, 16, 16
NW = NC * NS
ROWS_PER_W = R // NW
NVEC = N // L
G1 = 4
NG1 = NVEC // G1
NG2 = 8
GPH = NG1 // NG2
CAP = 1024

_NEG_INF = float("-inf")


def _splat_f(x):
    return jnp.broadcast_to(jnp.float32(x), (L,))


def _splat_i(x):
    return jnp.broadcast_to(jnp.int32(x), (L,))


@functools.partial(
    pl.kernel,
    out_type=(
        jax.ShapeDtypeStruct((R, KPAD), jnp.int32),
        jax.ShapeDtypeStruct((R, KPAD), jnp.float32),
    ),
    mesh=plsc.VectorSubcoreMesh(core_axis_name="c", subcore_axis_name="s"),
    compiler_params=pltpu.CompilerParams(needs_layout_passes=False),
    scratch_types=[
        pltpu.VMEM((N,), jnp.float32),
        pltpu.VMEM((N,), jnp.float32),
        pltpu.VMEM((NG1 * L,), jnp.float32),
        pltpu.VMEM((CAP,), jnp.float32),
        pltpu.VMEM((CAP,), jnp.int32),
        pltpu.VMEM((KPAD,), jnp.float32),
        pltpu.VMEM((KPAD,), jnp.int32),
        pltpu.SemaphoreType.DMA,
    ],
)
def _topk_sc(scores_hbm, oidx_hbm, ovals_hbm, row_a, row_b, gmax_v, cv_v,
             ci_v, sv_v, si_v, sem):
    lane = lax.iota(jnp.int32, L)

    def do_row(r, row_v):
        def h_body(h, tcur):
            def g_body(i, g2acc):
                g = h * GPH + i
                gm = row_v[pl.ds(g * G1 * L, L)]
                for j in range(1, G1):
                    gm = jnp.maximum(gm, row_v[pl.ds((g * G1 + j) * L, L)])
                gmax_v[pl.ds(g * L, L)] = gm
                return jnp.maximum(g2acc, gm)

            g2 = lax.fori_loop(0, GPH, g_body, _splat_f(-jnp.inf), unroll=4)

            def r_body(_, x):
                return jnp.where(x == jnp.max(x), jnp.float32(_NEG_INF), x)

            t_h = jnp.max(lax.fori_loop(0, 6, r_body, g2))
            return jnp.minimum(tcur, t_h)

        t = lax.fori_loop(0, NG2, h_body, jnp.float32(jnp.inf))

        def c_body(g, off):
            gm = gmax_v[pl.ds(g * L, L)]

            def append(off):
                def one(u, off2):
                    i = g * G1 + u
                    v = row_v[pl.ds(i * L, L)]
                    m = v >= t
                    cnt = jnp.sum(m.astype(jnp.int32))
                    pos = off2 + plsc.cumsum(m.astype(jnp.int32)) - 1
                    pos = jnp.minimum(pos, CAP - 1)
                    plsc.store_scatter(cv_v, [pos], v, mask=m)
                    plsc.store_scatter(ci_v, [pos], i * L + lane, mask=m)
                    return off2 + cnt

                for u in range(G1):
                    off = one(u, off)
                return off

            return lax.cond(jnp.any(gm >= t), append, lambda off: off, off)

        m_total = lax.fori_loop(0, NG1, c_body, 0, unroll=2)

        m_c = jnp.minimum(m_total, CAP)
        base = jnp.minimum((m_c // L) * L, CAP - L)
        tail = cv_v[pl.ds(base, L)]
        cv_v[pl.ds(base, L)] = jnp.where(lane >= m_c - base,
                                         jnp.float32(_NEG_INF), tail)
        nvc = (m_c + (L - 1)) // L

        def e_body(j, _):
            def mx_body(i, acc):
                return jnp.maximum(acc, cv_v[pl.ds(i * L, L)])

            mx = lax.fori_loop(0, nvc, mx_body, _splat_f(-jnp.inf))
            vstar = jnp.max(mx)

            def ix_body(i, acc):
                acc_idx, acc_pos = acc
                vv = cv_v[pl.ds(i * L, L)]
                ii = ci_v[pl.ds(i * L, L)]
                pp = i * L + lane
                better = (vv == vstar) & (ii < acc_idx)
                return (jnp.where(better, ii, acc_idx),
                        jnp.where(better, pp, acc_pos))

            acc_idx, acc_pos = lax.fori_loop(
                0, nvc, ix_body, (_splat_i(2**30), _splat_i(2**30)))
            istar = jnp.min(acc_idx)
            pstar = jnp.min(jnp.where(acc_idx == istar, acc_pos,
                                      jnp.int32(2**30)))
            lane0 = lane == 0
            plsc.store_scatter(sv_v, [_splat_i(0) + j],
                               jnp.broadcast_to(vstar, (L,)), mask=lane0)
            plsc.store_scatter(si_v, [_splat_i(0) + j],
                               jnp.broadcast_to(istar, (L,)), mask=lane0)
            plsc.store_scatter(cv_v, [_splat_i(0) + pstar],
                               _splat_f(-jnp.inf), mask=lane0)
            return 0

        lax.fori_loop(0, K, e_body, 0)

        pltpu.sync_copy(sv_v, ovals_hbm.at[r])
        pltpu.sync_copy(si_v, oidx_hbm.at[r])

    wid = lax.axis_index("s") * NC + lax.axis_index("c")
    r0 = wid * ROWS_PER_W
    pltpu.sync_copy(scores_hbm.at[r0], row_a)
    cp = pltpu.async_copy(scores_hbm.at[r0 + 1], row_b, sem)
    do_row(r0, row_a)
    cp.wait()
    do_row(r0 + 1, row_b)


def kernel(scores):
    idx_pad, vals_pad = _topk_sc(scores)
    return idx_pad[:, :K], vals_pad[:, :K]
